# baseline (speedup 1.0000x reference)
import functools

import jax
import jax.numpy as jnp
from jax import lax
from jax.experimental import pallas as pl
from jax.experimental.pallas import tpu as pltpu
from jax.experimental.pallas import tpu_sc as plsc


def _make_sc_tail(B, D, H, dtype):
    info = plsc.get_sparse_core_info()
    nw = info.num_cores * info.num_subcores
    rows = B - H
    r_per_w = rows // nw
    assert r_per_w * nw == rows

    mesh = plsc.VectorSubcoreMesh(core_axis_name="c", subcore_axis_name="s")

    @functools.partial(
        pl.kernel,
        out_type=jax.ShapeDtypeStruct((rows, D), dtype),
        mesh=mesh,
        scratch_types=[
            pltpu.VMEM((r_per_w, D), dtype),
            pltpu.SemaphoreType.DMA,
            pltpu.SemaphoreType.DMA,
        ],
    )
    def sc_tail(item_hbm, out_hbm, rows_v, sem_in, sem_out):
        wid = lax.axis_index("s") * info.num_cores + lax.axis_index("c")
        pltpu.async_copy(
            item_hbm.at[pl.ds(H + wid * r_per_w, r_per_w)], rows_v, sem_in
        ).wait()
        pltpu.async_copy(
            rows_v, out_hbm.at[pl.ds(wid * r_per_w, r_per_w)], sem_out
        ).wait()

    return sc_tail


def _tc_copy_body(item_ref, out_ref):
    out_ref[...] = item_ref[...]


def _make_tc_head(B, D, H, dtype, blk=2048):
    assert H % blk == 0
    return pl.pallas_call(
        _tc_copy_body,
        grid=(H // blk,),
        in_specs=[pl.BlockSpec((blk, D), lambda i: (i, 0))],
        out_specs=pl.BlockSpec((blk, D), lambda i: (i, 0)),
        out_shape=jax.ShapeDtypeStruct((B, D), dtype),
    )


def kernel(item, out, order_indices):
    B, D = item.shape
    H = 10240
    sc_part = _make_sc_tail(B, D, H, item.dtype)(item)
    tc_part = _make_tc_head(B, D, H, item.dtype)(item)
    return lax.dynamic_update_slice(tc_part, sc_part, (H, 0))


# speedup vs baseline: 1.0895x; 1.0895x over previous
import functools

import jax
import jax.numpy as jnp
from jax import lax
from jax.experimental import pallas as pl
from jax.experimental.pallas import tpu as pltpu
from jax.experimental.pallas import tpu_sc as plsc


def _make_queue_kernel(B, D, dtype):
    info = plsc.get_sparse_core_info()
    nw = info.num_cores * info.num_subcores
    b_per_w = B // nw
    assert b_per_w * nw == B
    half = b_per_w // 2

    mesh = plsc.VectorSubcoreMesh(core_axis_name="c", subcore_axis_name="s")

    @functools.partial(
        pl.kernel,
        out_type=jax.ShapeDtypeStruct((B, D), dtype),
        mesh=mesh,
        scratch_types=[
            pltpu.VMEM((b_per_w, D), dtype),
            pltpu.SemaphoreType.DMA,
            pltpu.SemaphoreType.DMA,
            pltpu.SemaphoreType.DMA,
            pltpu.SemaphoreType.DMA,
        ],
    )
    def queue_kernel(item_hbm, out_hbm, rows_v, si0, si1, so0, so1):
        wid = lax.axis_index("s") * info.num_cores + lax.axis_index("c")
        base = wid * b_per_w
        in0 = pltpu.async_copy(
            item_hbm.at[pl.ds(base, half)], rows_v.at[pl.ds(0, half)], si0
        )
        in1 = pltpu.async_copy(
            item_hbm.at[pl.ds(base + half, half)],
            rows_v.at[pl.ds(half, half)],
            si1,
        )
        in0.wait()
        out0 = pltpu.async_copy(
            rows_v.at[pl.ds(0, half)], out_hbm.at[pl.ds(base, half)], so0
        )
        in1.wait()
        out1 = pltpu.async_copy(
            rows_v.at[pl.ds(half, half)],
            out_hbm.at[pl.ds(base + half, half)],
            so1,
        )
        out0.wait()
        out1.wait()

    return queue_kernel


def kernel(item, out, order_indices):
    B, D = item.shape
    return _make_queue_kernel(B, D, item.dtype)(item)


# speedup vs baseline: 1.0967x; 1.0065x over previous
import functools

import jax
from jax import lax
from jax.experimental import pallas as pl
from jax.experimental.pallas import tpu as pltpu
from jax.experimental.pallas import tpu_sc as plsc


def _make_queue_kernel(B, D, dtype):
    info = plsc.get_sparse_core_info()
    nw = info.num_cores * info.num_subcores
    b_per_w = B // nw
    assert b_per_w * nw == B

    mesh = plsc.VectorSubcoreMesh(core_axis_name="c", subcore_axis_name="s")

    @functools.partial(
        pl.kernel,
        out_type=jax.ShapeDtypeStruct((B, D), dtype),
        mesh=mesh,
        scratch_types=[
            pltpu.VMEM((b_per_w, D), dtype),
            pltpu.SemaphoreType.DMA,
            pltpu.SemaphoreType.DMA,
        ],
    )
    def queue_kernel(item_hbm, out_hbm, rows_v, sem_in, sem_out):
        wid = lax.axis_index("s") * info.num_cores + lax.axis_index("c")
        base = wid * b_per_w
        pltpu.async_copy(
            item_hbm.at[pl.ds(base, b_per_w)], rows_v, sem_in
        ).wait()
        pltpu.async_copy(
            rows_v, out_hbm.at[pl.ds(base, b_per_w)], sem_out
        ).wait()

    return queue_kernel


def kernel(item, out, order_indices):
    B, D = item.shape
    return _make_queue_kernel(B, D, item.dtype)(item)
